# Initial kernel scaffold; baseline (speedup 1.0000x reference)
import jax
import jax.numpy as jnp
from jax.experimental import pallas as pl


def kernel(x1, edge_index1, x2, edge_index2, y, W11, b11, W12, b12, W21, b21, W22, b22, Wa1, ba1, Wa2, ba2, Wc1, bc1, Wc2, bc2):
    raise NotImplementedError("write your pallas kernel here")


# speedup vs baseline: 3.5882x; 3.5882x over previous
import functools

import jax
import jax.numpy as jnp
from jax import lax
from jax.experimental import pallas as pl
from jax.experimental.pallas import tpu as pltpu

N_NODES = 10000
IN_DIM = 128
GCN_H = 64
TEMPERATURE = 0.1
LAMBDA_CL = 0.1

INTERPRET = False


def _weights_body(y_ref, w_ref):
    y = y_ref[...].astype(jnp.float32)
    c1 = jnp.sum(y)
    c0 = jnp.float32(N_NODES) - c1
    cnt = jnp.where(y > 0.5, c1, c0)
    w_ref[...] = 1.0 / (cnt + 1e-10)


def _class_weights(y):
    return pl.pallas_call(
        _weights_body,
        out_shape=jax.ShapeDtypeStruct((N_NODES, 1), jnp.float32),
        interpret=INTERPRET,
    )(y.reshape(N_NODES, 1))


def _head_body(h1_ref, h2_ref, Wa1_ref, ba1_ref, Wa2_ref, ba2_ref,
               Wc1_ref, bc1_ref, Wc2_ref, bc2_ref, out_ref):
    h1 = h1_ref[...]
    h2 = h2_ref[...]
    combined = jnp.concatenate([h1, h2], axis=1)
    att = jnp.tanh(
        jnp.dot(combined, Wa1_ref[...], preferred_element_type=jnp.float32)
        + ba1_ref[...])
    logits_att = (
        jnp.dot(att, Wa2_ref[...], preferred_element_type=jnp.float32)
        + ba2_ref[...])
    aw = jax.nn.softmax(logits_att, axis=1)
    fused = jnp.concatenate([h1 * aw[:, 0:1], h2 * aw[:, 1:2]], axis=1)
    hid = jax.nn.relu(
        jnp.dot(fused, Wc1_ref[...], preferred_element_type=jnp.float32)
        + bc1_ref[...])
    out = (jnp.dot(hid, Wc2_ref[...], preferred_element_type=jnp.float32)
           + bc2_ref[...])
    out_ref[...] = jax.nn.log_softmax(out, axis=1)


def _head(h1, h2, Wa1, ba1, Wa2, ba2, Wc1, bc1, Wc2, bc2):
    R = 2000
    grid = (N_NODES // R,)
    row_spec = pl.BlockSpec((R, GCN_H), lambda i: (i, 0))
    full = lambda shape: pl.BlockSpec(shape, lambda i: tuple(0 for _ in shape))
    return pl.pallas_call(
        _head_body,
        grid=grid,
        in_specs=[
            row_spec, row_spec,
            full(Wa1.shape), full((1, 32)),
            full(Wa2.shape), full((1, 2)),
            full(Wc1.shape), full((1, 16)),
            full(Wc2.shape), full((1, 2)),
        ],
        out_specs=pl.BlockSpec((R, 2), lambda i: (i, 0)),
        out_shape=jax.ShapeDtypeStruct((N_NODES, 2), jnp.float32),
        interpret=INTERPRET,
    )(h1, h2, Wa1, ba1.reshape(1, -1), Wa2, ba2.reshape(1, -1),
      Wc1, bc1.reshape(1, -1), Wc2, bc2.reshape(1, -1))


_LR = 400
_LC = 2000


def _loss_body(h1r_ref, h2c_ref, h2r_ref, w_ref, out_ref, m_ref, a_ref, s_ref):
    i = pl.program_id(0)
    j = pl.program_id(1)
    nj = pl.num_programs(1)

    @pl.when(jnp.logical_and(i == 0, j == 0))
    def _():
        s_ref[0] = jnp.float32(0.0)

    @pl.when(j == 0)
    def _():
        m_ref[...] = jnp.full((_LR, 1), -1e30, jnp.float32)
        a_ref[...] = jnp.zeros((_LR, 1), jnp.float32)

    h1r = h1r_ref[...]
    w = w_ref[...]
    scale = w / TEMPERATURE
    sim = jax.lax.dot_general(
        h1r, h2c_ref[...], (((1,), (1,)), ((), ())),
        preferred_element_type=jnp.float32)
    wl = sim * scale
    rows = jax.lax.broadcasted_iota(jnp.int32, (_LR, _LC), 0) + i * _LR
    cols = jax.lax.broadcasted_iota(jnp.int32, (_LR, _LC), 1) + j * _LC
    wl = jnp.where(rows == cols, 0.0, wl)

    m_prev = m_ref[...]
    a_prev = a_ref[...]
    m_new = jnp.maximum(m_prev, jnp.max(wl, axis=1, keepdims=True))
    a_new = a_prev * jnp.exp(m_prev - m_new) + jnp.sum(
        jnp.exp(wl - m_new), axis=1, keepdims=True)
    m_ref[...] = m_new
    a_ref[...] = a_new

    @pl.when(j == nj - 1)
    def _():
        h2r = h2r_ref[...]
        n1 = jnp.maximum(
            jnp.sqrt(jnp.sum(h1r * h1r, axis=1, keepdims=True)), 1e-8)
        n2 = jnp.maximum(
            jnp.sqrt(jnp.sum(h2r * h2r, axis=1, keepdims=True)), 1e-8)
        pos = jnp.sum(h1r * h2r, axis=1, keepdims=True) / (n1 * n2)
        wpos = pos / TEMPERATURE * w
        m_f = jnp.maximum(m_ref[...], wpos)
        a_f = a_ref[...] * jnp.exp(m_ref[...] - m_f) + jnp.exp(wpos - m_f)
        lse = jnp.log(a_f) + m_f
        s_ref[0] += jnp.sum(lse - wpos)

        @pl.when(i == pl.num_programs(0) - 1)
        def _():
            out_ref[0, 0] = s_ref[0] / N_NODES * LAMBDA_CL


def _contrastive_loss(h1, h2, w):
    grid = (N_NODES // _LR, N_NODES // _LC)
    out = pl.pallas_call(
        _loss_body,
        grid=grid,
        in_specs=[
            pl.BlockSpec((_LR, GCN_H), lambda i, j: (i, 0)),
            pl.BlockSpec((_LC, GCN_H), lambda i, j: (j, 0)),
            pl.BlockSpec((_LR, GCN_H), lambda i, j: (i, 0)),
            pl.BlockSpec((_LR, 1), lambda i, j: (i, 0)),
        ],
        out_specs=pl.BlockSpec(memory_space=pltpu.SMEM),
        out_shape=jax.ShapeDtypeStruct((1, 1), jnp.float32),
        scratch_shapes=[
            pltpu.VMEM((_LR, 1), jnp.float32),
            pltpu.VMEM((_LR, 1), jnp.float32),
            pltpu.SMEM((1,), jnp.float32),
        ],
        interpret=INTERPRET,
    )(h1, h2, h2, w)
    return out[0, 0]


def _gcn(x, src, dst, dinv, W, b):
    h = (x @ W) * dinv[:, None]
    agg = jax.ops.segment_sum(h[src], dst, num_segments=N_NODES)
    return jax.nn.relu((agg + h) * dinv[:, None] + b)


def kernel(x1, edge_index1, x2, edge_index2, y,
           W11, b11, W12, b12, W21, b21, W22, b22,
           Wa1, ba1, Wa2, ba2, Wc1, bc1, Wc2, bc2):
    src1, dst1 = edge_index1[0], edge_index1[1]
    src2, dst2 = edge_index2[0], edge_index2[1]
    ones = jnp.ones((src1.shape[0],), jnp.float32)
    deg1 = jax.ops.segment_sum(ones, dst1, num_segments=N_NODES) + 1.0
    deg2 = jax.ops.segment_sum(ones, dst2, num_segments=N_NODES) + 1.0
    dinv1 = jax.lax.rsqrt(deg1)
    dinv2 = jax.lax.rsqrt(deg2)

    g1 = _gcn(x1, src1, dst1, dinv1, W11, b11)
    g1 = _gcn(g1, src1, dst1, dinv1, W12, b12)
    g2 = _gcn(x2, src2, dst2, dinv2, W21, b21)
    g2 = _gcn(g2, src2, dst2, dinv2, W22, b22)

    log_probs = _head(g1, g2, Wa1, ba1, Wa2, ba2, Wc1, bc1, Wc2, bc2)
    w = _class_weights(y)
    loss = _contrastive_loss(g1, g2, w)
    return (log_probs, loss)


# speedup vs baseline: 14.5442x; 4.0534x over previous
import functools

import jax
import jax.numpy as jnp
from jax import lax
from jax.experimental import pallas as pl
from jax.experimental.pallas import tpu as pltpu
from jax.experimental.pallas import tpu_sc as plsc

N_NODES = 10000
IN_DIM = 128
GCN_H = 64
TEMPERATURE = 0.1
LAMBDA_CL = 0.1

N_EDGES = 320000
N_PAD = 10240
RPT = N_PAD // 16
DUMMY = N_PAD - 1
EK = 512
NCHUNK = 40
EPT = EK * NCHUNK
E_PAD = EPT * 16

INTERPRET = False


@functools.cache
def _get_mesh():
    return plsc.VectorSubcoreMesh(core_axis_name="c", subcore_axis_name="s",
                                  num_cores=2, num_subcores=16)


def _sc_counts_body(se_hbm, zeros_hbm, ones_hbm, out_hbm, dstv, onesv, acc_sh):
    cid = lax.axis_index("c")
    sid = lax.axis_index("s")
    pltpu.sync_copy(zeros_hbm.at[pl.ds(sid * RPT, RPT)],
                    acc_sh.at[pl.ds(sid * RPT, RPT)])
    pltpu.sync_copy(ones_hbm, onesv)
    plsc.subcore_barrier()

    @pl.loop(0, NCHUNK)
    def _(c):
        base = sid * EPT + c * EK
        pltpu.sync_copy(se_hbm.at[cid, 1, pl.ds(base, EK)], dstv)
        pltpu.sync_copy(onesv, acc_sh.at[dstv], add=True)

    plsc.subcore_barrier()
    pltpu.sync_copy(acc_sh.at[pl.ds(sid * RPT, RPT)],
                    out_hbm.at[cid, pl.ds(sid * RPT, RPT)])


@functools.cache
def _sc_counts_kernel():
    return pl.kernel(
        _sc_counts_body,
        out_type=jax.ShapeDtypeStruct((2, N_PAD, 16), jnp.float32),
        mesh=_get_mesh(),
        scratch_types=[
            pltpu.VMEM((EK,), jnp.int32),
            pltpu.VMEM((EK, 16), jnp.float32),
            pltpu.VMEM_SHARED((N_PAD, 16), jnp.float32),
        ],
        compiler_params=pltpu.CompilerParams(use_tc_tiling_on_sc=False),
    )


def _sc_counts(se, zeros16, ones16):
    return _sc_counts_kernel()(se, zeros16, ones16)


def _sc_conv_body(h1_hbm, h2_hbm, se_hbm, out_hbm, srcv, dstv, rows, acc_sh,
                  sem):
    cid = lax.axis_index("c")
    sid = lax.axis_index("s")

    def channel(h_hbm, ch):
        pltpu.sync_copy(h_hbm.at[pl.ds(sid * RPT, RPT)],
                        acc_sh.at[pl.ds(sid * RPT, RPT)])
        plsc.subcore_barrier()

        @pl.loop(0, NCHUNK)
        def _(c):
            base = sid * EPT + c * EK
            pltpu.sync_copy(se_hbm.at[ch, 0, pl.ds(base, EK)], srcv)
            pltpu.sync_copy(se_hbm.at[ch, 1, pl.ds(base, EK)], dstv)
            pltpu.async_copy(h_hbm.at[srcv], rows, sem).wait()
            pltpu.sync_copy(rows, acc_sh.at[dstv], add=True)

        plsc.subcore_barrier()
        pltpu.sync_copy(acc_sh.at[pl.ds(sid * RPT, RPT)],
                        out_hbm.at[ch, pl.ds(sid * RPT, RPT)])

    @pl.when(cid == 0)
    def _():
        channel(h1_hbm, 0)

    @pl.when(cid == 1)
    def _():
        channel(h2_hbm, 1)


@functools.cache
def _sc_conv_kernel():
    return pl.kernel(
        _sc_conv_body,
        out_type=jax.ShapeDtypeStruct((2, N_PAD, GCN_H), jnp.float32),
        mesh=_get_mesh(),
        scratch_types=[
            pltpu.VMEM((EK,), jnp.int32),
            pltpu.VMEM((EK,), jnp.int32),
            pltpu.VMEM((EK, GCN_H), jnp.float32),
            pltpu.VMEM_SHARED((N_PAD, GCN_H), jnp.float32),
            pltpu.SemaphoreType.DMA,
        ],
        compiler_params=pltpu.CompilerParams(use_tc_tiling_on_sc=False),
    )


def _sc_conv(h1p, h2p, se):
    return _sc_conv_kernel()(h1p, h2p, se)


_TCR = 640


def _row_mask(i, blk):
    rows = jax.lax.broadcasted_iota(jnp.int32, blk, 0) + i * _TCR
    return rows < N_NODES


def _prep_body(c1_ref, c2_ref, x1_ref, x2_ref, W11_ref, W21_ref,
               h1_ref, h2_ref):
    i = pl.program_id(0)
    mask = _row_mask(i, (_TCR, GCN_H))
    dinv1 = jax.lax.rsqrt(c1_ref[0, :, 0:1] + 1.0)
    dinv2 = jax.lax.rsqrt(c2_ref[0, :, 0:1] + 1.0)
    h1 = jnp.dot(x1_ref[...], W11_ref[...],
                 preferred_element_type=jnp.float32) * dinv1
    h2 = jnp.dot(x2_ref[...], W21_ref[...],
                 preferred_element_type=jnp.float32) * dinv2
    h1_ref[...] = jnp.where(mask, h1, 0.0)
    h2_ref[...] = jnp.where(mask, h2, 0.0)


def _mid_body(c1_ref, c2_ref, s1_ref, s2_ref, b11_ref, b21_ref,
              W12_ref, W22_ref, h1_ref, h2_ref):
    i = pl.program_id(0)
    mask = _row_mask(i, (_TCR, GCN_H))
    dinv1 = jax.lax.rsqrt(c1_ref[0, :, 0:1] + 1.0)
    dinv2 = jax.lax.rsqrt(c2_ref[0, :, 0:1] + 1.0)
    g1 = jax.nn.relu(s1_ref[0] * dinv1 + b11_ref[...])
    g2 = jax.nn.relu(s2_ref[0] * dinv2 + b21_ref[...])
    h1 = jnp.dot(g1, W12_ref[...], preferred_element_type=jnp.float32) * dinv1
    h2 = jnp.dot(g2, W22_ref[...], preferred_element_type=jnp.float32) * dinv2
    h1_ref[...] = jnp.where(mask, h1, 0.0)
    h2_ref[...] = jnp.where(mask, h2, 0.0)


def _fin_body(c1_ref, c2_ref, s1_ref, s2_ref, b12_ref, b22_ref,
              Wa1_ref, ba1_ref, Wa2_ref, ba2_ref,
              Wc1_ref, bc1_ref, Wc2_ref, bc2_ref,
              g1_ref, g2_ref, lp_ref):
    dinv1 = jax.lax.rsqrt(c1_ref[0, :, 0:1] + 1.0)
    dinv2 = jax.lax.rsqrt(c2_ref[0, :, 0:1] + 1.0)
    g1 = jax.nn.relu(s1_ref[0] * dinv1 + b12_ref[...])
    g2 = jax.nn.relu(s2_ref[0] * dinv2 + b22_ref[...])
    g1_ref[...] = g1
    g2_ref[...] = g2
    combined = jnp.concatenate([g1, g2], axis=1)
    att = jnp.tanh(
        jnp.dot(combined, Wa1_ref[...], preferred_element_type=jnp.float32)
        + ba1_ref[...])
    logits_att = (
        jnp.dot(att, Wa2_ref[...], preferred_element_type=jnp.float32)
        + ba2_ref[...])
    aw = jax.nn.softmax(logits_att, axis=1)
    fused = jnp.concatenate([g1 * aw[:, 0:1], g2 * aw[:, 1:2]], axis=1)
    hid = jax.nn.relu(
        jnp.dot(fused, Wc1_ref[...], preferred_element_type=jnp.float32)
        + bc1_ref[...])
    out = (jnp.dot(hid, Wc2_ref[...], preferred_element_type=jnp.float32)
           + bc2_ref[...])
    lp_ref[...] = jax.nn.log_softmax(out, axis=1)


def _weights_body(y_ref, w_ref):
    y = y_ref[...].astype(jnp.float32)
    c1 = jnp.sum(y)
    c0 = jnp.float32(N_NODES) - c1
    cnt = jnp.where(y > 0.5, c1, c0)
    w_ref[...] = 1.0 / (cnt + 1e-10)


_LR = 400
_LC = 2000


def _loss_body(h1r_ref, h2c_ref, h2r_ref, w_ref, out_ref, m_ref, a_ref, s_ref):
    i = pl.program_id(0)
    j = pl.program_id(1)
    nj = pl.num_programs(1)

    @pl.when(jnp.logical_and(i == 0, j == 0))
    def _():
        s_ref[0] = jnp.float32(0.0)

    @pl.when(j == 0)
    def _():
        m_ref[...] = jnp.full((_LR, 1), -1e30, jnp.float32)
        a_ref[...] = jnp.zeros((_LR, 1), jnp.float32)

    h1r = h1r_ref[...]
    w = w_ref[...]
    scale = w / TEMPERATURE
    sim = jax.lax.dot_general(
        h1r, h2c_ref[...], (((1,), (1,)), ((), ())),
        preferred_element_type=jnp.float32)
    wl = sim * scale
    rows = jax.lax.broadcasted_iota(jnp.int32, (_LR, _LC), 0) + i * _LR
    cols = jax.lax.broadcasted_iota(jnp.int32, (_LR, _LC), 1) + j * _LC
    wl = jnp.where(rows == cols, 0.0, wl)

    m_prev = m_ref[...]
    a_prev = a_ref[...]
    m_new = jnp.maximum(m_prev, jnp.max(wl, axis=1, keepdims=True))
    a_new = a_prev * jnp.exp(m_prev - m_new) + jnp.sum(
        jnp.exp(wl - m_new), axis=1, keepdims=True)
    m_ref[...] = m_new
    a_ref[...] = a_new

    @pl.when(j == nj - 1)
    def _():
        h2r = h2r_ref[...]
        n1 = jnp.maximum(
            jnp.sqrt(jnp.sum(h1r * h1r, axis=1, keepdims=True)), 1e-8)
        n2 = jnp.maximum(
            jnp.sqrt(jnp.sum(h2r * h2r, axis=1, keepdims=True)), 1e-8)
        pos = jnp.sum(h1r * h2r, axis=1, keepdims=True) / (n1 * n2)
        wpos = pos / TEMPERATURE * w
        m_f = jnp.maximum(m_ref[...], wpos)
        a_f = a_ref[...] * jnp.exp(m_ref[...] - m_f) + jnp.exp(wpos - m_f)
        lse = jnp.log(a_f) + m_f
        s_ref[0] += jnp.sum(lse - wpos)

        @pl.when(i == pl.num_programs(0) - 1)
        def _():
            out_ref[0, 0] = s_ref[0] / N_NODES * LAMBDA_CL


def _contrastive_loss(h1, h2, w):
    grid = (N_NODES // _LR, N_NODES // _LC)
    out = pl.pallas_call(
        _loss_body,
        grid=grid,
        in_specs=[
            pl.BlockSpec((_LR, GCN_H), lambda i, j: (i, 0)),
            pl.BlockSpec((_LC, GCN_H), lambda i, j: (j, 0)),
            pl.BlockSpec((_LR, GCN_H), lambda i, j: (i, 0)),
            pl.BlockSpec((_LR, 1), lambda i, j: (i, 0)),
        ],
        out_specs=pl.BlockSpec(memory_space=pltpu.SMEM),
        out_shape=jax.ShapeDtypeStruct((1, 1), jnp.float32),
        scratch_shapes=[
            pltpu.VMEM((_LR, 1), jnp.float32),
            pltpu.VMEM((_LR, 1), jnp.float32),
            pltpu.SMEM((1,), jnp.float32),
        ],
        interpret=INTERPRET,
    )(h1, h2, h2, w)
    return out[0, 0]


def _cnt_spec(ch):
    return pl.BlockSpec((1, _TCR, 16), lambda i, ch=ch: (ch, i, 0))


def _s_spec(ch):
    return pl.BlockSpec((1, _TCR, GCN_H), lambda i, ch=ch: (ch, i, 0))


def _full(shape):
    return pl.BlockSpec(shape, lambda i: tuple(0 for _ in shape))


_row64 = pl.BlockSpec((_TCR, GCN_H), lambda i: (i, 0))
_grid16 = (N_PAD // _TCR,)


def _prep(cnt, x1, x2, W11, W21):
    return pl.pallas_call(
        _prep_body,
        grid=_grid16,
        in_specs=[
            _cnt_spec(0), _cnt_spec(1),
            pl.BlockSpec((_TCR, IN_DIM), lambda i: (i, 0)),
            pl.BlockSpec((_TCR, IN_DIM), lambda i: (i, 0)),
            _full((IN_DIM, GCN_H)), _full((IN_DIM, GCN_H)),
        ],
        out_specs=[_row64, _row64],
        out_shape=[jax.ShapeDtypeStruct((N_PAD, GCN_H), jnp.float32)] * 2,
        interpret=INTERPRET,
    )(cnt, cnt, x1, x2, W11, W21)


def _mid(cnt, s, b11, b21, W12, W22):
    return pl.pallas_call(
        _mid_body,
        grid=_grid16,
        in_specs=[
            _cnt_spec(0), _cnt_spec(1), _s_spec(0), _s_spec(1),
            _full((1, GCN_H)), _full((1, GCN_H)),
            _full((GCN_H, GCN_H)), _full((GCN_H, GCN_H)),
        ],
        out_specs=[_row64, _row64],
        out_shape=[jax.ShapeDtypeStruct((N_PAD, GCN_H), jnp.float32)] * 2,
        interpret=INTERPRET,
    )(cnt, cnt, s, s, b11.reshape(1, -1), b21.reshape(1, -1), W12, W22)


def _fin(cnt, s, b12, b22, Wa1, ba1, Wa2, ba2, Wc1, bc1, Wc2, bc2):
    return pl.pallas_call(
        _fin_body,
        grid=_grid16,
        in_specs=[
            _cnt_spec(0), _cnt_spec(1), _s_spec(0), _s_spec(1),
            _full((1, GCN_H)), _full((1, GCN_H)),
            _full((2 * GCN_H, 32)), _full((1, 32)),
            _full((32, 2)), _full((1, 2)),
            _full((2 * GCN_H, 16)), _full((1, 16)),
            _full((16, 2)), _full((1, 2)),
        ],
        out_specs=[_row64, _row64, pl.BlockSpec((_TCR, 2), lambda i: (i, 0))],
        out_shape=[
            jax.ShapeDtypeStruct((N_NODES, GCN_H), jnp.float32),
            jax.ShapeDtypeStruct((N_NODES, GCN_H), jnp.float32),
            jax.ShapeDtypeStruct((N_NODES, 2), jnp.float32),
        ],
        interpret=INTERPRET,
    )(cnt, cnt, s, s, b12.reshape(1, -1), b22.reshape(1, -1),
      Wa1, ba1.reshape(1, -1), Wa2, ba2.reshape(1, -1),
      Wc1, bc1.reshape(1, -1), Wc2, bc2.reshape(1, -1))


def _class_weights(y):
    return pl.pallas_call(
        _weights_body,
        out_shape=jax.ShapeDtypeStruct((N_NODES, 1), jnp.float32),
        interpret=INTERPRET,
    )(y.reshape(N_NODES, 1))


def _pad_edges(edge_index):
    pad = jnp.full((E_PAD - N_EDGES,), DUMMY, jnp.int32)
    src = jnp.concatenate([edge_index[0].astype(jnp.int32), pad])
    dst = jnp.concatenate([edge_index[1].astype(jnp.int32), pad])
    return jnp.stack([src, dst])


def kernel(x1, edge_index1, x2, edge_index2, y,
           W11, b11, W12, b12, W21, b21, W22, b22,
           Wa1, ba1, Wa2, ba2, Wc1, bc1, Wc2, bc2):
    se = jnp.stack([_pad_edges(edge_index1), _pad_edges(edge_index2)])
    zeros16 = jnp.zeros((N_PAD, 16), jnp.float32)
    ones16 = jnp.ones((EK, 16), jnp.float32)

    cnt = _sc_counts(se, zeros16, ones16)
    h1p, h2p = _prep(cnt, x1, x2, W11, W21)
    s_l1 = _sc_conv(h1p, h2p, se)
    h1q, h2q = _mid(cnt, s_l1, b11, b21, W12, W22)
    s_l2 = _sc_conv(h1q, h2q, se)
    g1, g2, log_probs = _fin(cnt, s_l2, b12, b22,
                             Wa1, ba1, Wa2, ba2, Wc1, bc1, Wc2, bc2)
    w = _class_weights(y)
    loss = _contrastive_loss(g1, g2, w)
    return (log_probs, loss)


# speedup vs baseline: 16.3404x; 1.1235x over previous
import functools

import jax
import jax.numpy as jnp
from jax import lax
from jax.experimental import pallas as pl
from jax.experimental.pallas import tpu as pltpu
from jax.experimental.pallas import tpu_sc as plsc

N_NODES = 10000
IN_DIM = 128
GCN_H = 64
TEMPERATURE = 0.1
LAMBDA_CL = 0.1

N_EDGES = 320000
N_PAD = 10240
RPT = N_PAD // 16
DUMMY = N_PAD - 1
EK = 512
NCHUNK = 40
EPT = EK * NCHUNK
E_PAD = EPT * 16

INTERPRET = False


@functools.cache
def _get_mesh():
    return plsc.VectorSubcoreMesh(core_axis_name="c", subcore_axis_name="s",
                                  num_cores=2, num_subcores=16)


def _sc_counts_body(se_hbm, zeros_hbm, ones_hbm, out_hbm, dstv, onesv, acc_sh):
    cid = lax.axis_index("c")
    sid = lax.axis_index("s")
    pltpu.sync_copy(zeros_hbm.at[pl.ds(sid * RPT, RPT)],
                    acc_sh.at[pl.ds(sid * RPT, RPT)])
    pltpu.sync_copy(ones_hbm, onesv)
    plsc.subcore_barrier()

    @pl.loop(0, NCHUNK)
    def _(c):
        base = sid * EPT + c * EK
        pltpu.sync_copy(se_hbm.at[cid, 1, pl.ds(base, EK)], dstv)
        pltpu.sync_copy(onesv, acc_sh.at[dstv], add=True)

    plsc.subcore_barrier()
    pltpu.sync_copy(acc_sh.at[pl.ds(sid * RPT, RPT)],
                    out_hbm.at[cid, pl.ds(sid * RPT, RPT)])


@functools.cache
def _sc_counts_kernel():
    return pl.kernel(
        _sc_counts_body,
        out_type=jax.ShapeDtypeStruct((2, N_PAD, 16), jnp.float32),
        mesh=_get_mesh(),
        scratch_types=[
            pltpu.VMEM((EK,), jnp.int32),
            pltpu.VMEM((EK, 16), jnp.float32),
            pltpu.VMEM_SHARED((N_PAD, 16), jnp.float32),
        ],
        compiler_params=pltpu.CompilerParams(use_tc_tiling_on_sc=False),
    )


def _sc_counts(se, zeros16, ones16):
    return _sc_counts_kernel()(se, zeros16, ones16)


def _sc_conv_body(h1_hbm, h2_hbm, se_hbm, out_hbm,
                  srcv0, dstv0, rows0, srcv1, dstv1, rows1, acc_sh,
                  sem0, sem1):
    cid = lax.axis_index("c")
    sid = lax.axis_index("s")
    srcvs = (srcv0, srcv1)
    dstvs = (dstv0, dstv1)
    rowss = (rows0, rows1)
    sems = (sem0, sem1)

    def channel(h_hbm, ch):
        base0 = sid * EPT
        pltpu.sync_copy(se_hbm.at[ch, 0, pl.ds(base0, EK)], srcv0)
        pltpu.sync_copy(se_hbm.at[ch, 1, pl.ds(base0, EK)], dstv0)
        gather0 = pltpu.async_copy(h_hbm.at[srcv0], rows0, sem0)
        pltpu.sync_copy(h_hbm.at[pl.ds(sid * RPT, RPT)],
                        acc_sh.at[pl.ds(sid * RPT, RPT)])
        plsc.subcore_barrier()
        del gather0

        @pl.loop(0, NCHUNK // 2)
        def _(cb):
            for p in (0, 1):
                c = cb * 2 + p
                q = 1 - p

                @pl.when(c + 1 < NCHUNK)
                def _():
                    base = sid * EPT + (c + 1) * EK
                    pltpu.sync_copy(se_hbm.at[ch, 0, pl.ds(base, EK)],
                                    srcvs[q])
                    pltpu.sync_copy(se_hbm.at[ch, 1, pl.ds(base, EK)],
                                    dstvs[q])
                    pltpu.async_copy(h_hbm.at[srcvs[q]], rowss[q], sems[q])

                pltpu.make_async_copy(h_hbm.at[srcvs[p]], rowss[p],
                                      sems[p]).wait()
                pltpu.sync_copy(rowss[p], acc_sh.at[dstvs[p]], add=True)

        plsc.subcore_barrier()
        pltpu.sync_copy(acc_sh.at[pl.ds(sid * RPT, RPT)],
                        out_hbm.at[ch, pl.ds(sid * RPT, RPT)])

    @pl.when(cid == 0)
    def _():
        channel(h1_hbm, 0)

    @pl.when(cid == 1)
    def _():
        channel(h2_hbm, 1)


@functools.cache
def _sc_conv_kernel():
    return pl.kernel(
        _sc_conv_body,
        out_type=jax.ShapeDtypeStruct((2, N_PAD, GCN_H), jnp.float32),
        mesh=_get_mesh(),
        scratch_types=[
            pltpu.VMEM((EK,), jnp.int32),
            pltpu.VMEM((EK,), jnp.int32),
            pltpu.VMEM((EK, GCN_H), jnp.float32),
            pltpu.VMEM((EK,), jnp.int32),
            pltpu.VMEM((EK,), jnp.int32),
            pltpu.VMEM((EK, GCN_H), jnp.float32),
            pltpu.VMEM_SHARED((N_PAD, GCN_H), jnp.float32),
            pltpu.SemaphoreType.DMA,
            pltpu.SemaphoreType.DMA,
        ],
        compiler_params=pltpu.CompilerParams(use_tc_tiling_on_sc=False),
    )


def _sc_conv(h1p, h2p, se):
    return _sc_conv_kernel()(h1p, h2p, se)


_TCR = 640


def _row_mask(i, blk):
    rows = jax.lax.broadcasted_iota(jnp.int32, blk, 0) + i * _TCR
    return rows < N_NODES


def _prep_body(c1_ref, c2_ref, x1_ref, x2_ref, W11_ref, W21_ref,
               h1_ref, h2_ref):
    i = pl.program_id(0)
    mask = _row_mask(i, (_TCR, GCN_H))
    dinv1 = jax.lax.rsqrt(c1_ref[0, :, 0:1] + 1.0)
    dinv2 = jax.lax.rsqrt(c2_ref[0, :, 0:1] + 1.0)
    h1 = jnp.dot(x1_ref[...], W11_ref[...],
                 preferred_element_type=jnp.float32) * dinv1
    h2 = jnp.dot(x2_ref[...], W21_ref[...],
                 preferred_element_type=jnp.float32) * dinv2
    h1_ref[...] = jnp.where(mask, h1, 0.0)
    h2_ref[...] = jnp.where(mask, h2, 0.0)


def _mid_body(c1_ref, c2_ref, s1_ref, s2_ref, b11_ref, b21_ref,
              W12_ref, W22_ref, h1_ref, h2_ref):
    i = pl.program_id(0)
    mask = _row_mask(i, (_TCR, GCN_H))
    dinv1 = jax.lax.rsqrt(c1_ref[0, :, 0:1] + 1.0)
    dinv2 = jax.lax.rsqrt(c2_ref[0, :, 0:1] + 1.0)
    g1 = jax.nn.relu(s1_ref[0] * dinv1 + b11_ref[...])
    g2 = jax.nn.relu(s2_ref[0] * dinv2 + b21_ref[...])
    h1 = jnp.dot(g1, W12_ref[...], preferred_element_type=jnp.float32) * dinv1
    h2 = jnp.dot(g2, W22_ref[...], preferred_element_type=jnp.float32) * dinv2
    h1_ref[...] = jnp.where(mask, h1, 0.0)
    h2_ref[...] = jnp.where(mask, h2, 0.0)


def _fin_body(c1_ref, c2_ref, s1_ref, s2_ref, b12_ref, b22_ref,
              Wa1_ref, ba1_ref, Wa2_ref, ba2_ref,
              Wc1_ref, bc1_ref, Wc2_ref, bc2_ref,
              g1_ref, g2_ref, lp_ref):
    dinv1 = jax.lax.rsqrt(c1_ref[0, :, 0:1] + 1.0)
    dinv2 = jax.lax.rsqrt(c2_ref[0, :, 0:1] + 1.0)
    g1 = jax.nn.relu(s1_ref[0] * dinv1 + b12_ref[...])
    g2 = jax.nn.relu(s2_ref[0] * dinv2 + b22_ref[...])
    g1_ref[...] = g1
    g2_ref[...] = g2
    combined = jnp.concatenate([g1, g2], axis=1)
    att = jnp.tanh(
        jnp.dot(combined, Wa1_ref[...], preferred_element_type=jnp.float32)
        + ba1_ref[...])
    logits_att = (
        jnp.dot(att, Wa2_ref[...], preferred_element_type=jnp.float32)
        + ba2_ref[...])
    aw = jax.nn.softmax(logits_att, axis=1)
    fused = jnp.concatenate([g1 * aw[:, 0:1], g2 * aw[:, 1:2]], axis=1)
    hid = jax.nn.relu(
        jnp.dot(fused, Wc1_ref[...], preferred_element_type=jnp.float32)
        + bc1_ref[...])
    out = (jnp.dot(hid, Wc2_ref[...], preferred_element_type=jnp.float32)
           + bc2_ref[...])
    lp_ref[...] = jax.nn.log_softmax(out, axis=1)


def _weights_body(y_ref, w_ref):
    y = y_ref[...].astype(jnp.float32)
    c1 = jnp.sum(y)
    c0 = jnp.float32(N_NODES) - c1
    cnt = jnp.where(y > 0.5, c1, c0)
    w_ref[...] = 1.0 / (cnt + 1e-10)


_LR = 400
_LC = 2000


def _loss_body(h1r_ref, h2c_ref, h2r_ref, w_ref, out_ref, m_ref, a_ref, s_ref):
    i = pl.program_id(0)
    j = pl.program_id(1)
    nj = pl.num_programs(1)

    @pl.when(jnp.logical_and(i == 0, j == 0))
    def _():
        s_ref[0] = jnp.float32(0.0)

    @pl.when(j == 0)
    def _():
        m_ref[...] = jnp.full((_LR, 1), -1e30, jnp.float32)
        a_ref[...] = jnp.zeros((_LR, 1), jnp.float32)

    h1r = h1r_ref[...]
    w = w_ref[...]
    scale = w / TEMPERATURE
    sim = jax.lax.dot_general(
        h1r, h2c_ref[...], (((1,), (1,)), ((), ())),
        preferred_element_type=jnp.float32)
    wl = sim * scale
    rows = jax.lax.broadcasted_iota(jnp.int32, (_LR, _LC), 0) + i * _LR
    cols = jax.lax.broadcasted_iota(jnp.int32, (_LR, _LC), 1) + j * _LC
    wl = jnp.where(rows == cols, 0.0, wl)

    m_prev = m_ref[...]
    a_prev = a_ref[...]
    m_new = jnp.maximum(m_prev, jnp.max(wl, axis=1, keepdims=True))
    a_new = a_prev * jnp.exp(m_prev - m_new) + jnp.sum(
        jnp.exp(wl - m_new), axis=1, keepdims=True)
    m_ref[...] = m_new
    a_ref[...] = a_new

    @pl.when(j == nj - 1)
    def _():
        h2r = h2r_ref[...]
        n1 = jnp.maximum(
            jnp.sqrt(jnp.sum(h1r * h1r, axis=1, keepdims=True)), 1e-8)
        n2 = jnp.maximum(
            jnp.sqrt(jnp.sum(h2r * h2r, axis=1, keepdims=True)), 1e-8)
        pos = jnp.sum(h1r * h2r, axis=1, keepdims=True) / (n1 * n2)
        wpos = pos / TEMPERATURE * w
        m_f = jnp.maximum(m_ref[...], wpos)
        a_f = a_ref[...] * jnp.exp(m_ref[...] - m_f) + jnp.exp(wpos - m_f)
        lse = jnp.log(a_f) + m_f
        s_ref[0] += jnp.sum(lse - wpos)

        @pl.when(i == pl.num_programs(0) - 1)
        def _():
            out_ref[0, 0] = s_ref[0] / N_NODES * LAMBDA_CL


def _contrastive_loss(h1, h2, w):
    grid = (N_NODES // _LR, N_NODES // _LC)
    out = pl.pallas_call(
        _loss_body,
        grid=grid,
        in_specs=[
            pl.BlockSpec((_LR, GCN_H), lambda i, j: (i, 0)),
            pl.BlockSpec((_LC, GCN_H), lambda i, j: (j, 0)),
            pl.BlockSpec((_LR, GCN_H), lambda i, j: (i, 0)),
            pl.BlockSpec((_LR, 1), lambda i, j: (i, 0)),
        ],
        out_specs=pl.BlockSpec(memory_space=pltpu.SMEM),
        out_shape=jax.ShapeDtypeStruct((1, 1), jnp.float32),
        scratch_shapes=[
            pltpu.VMEM((_LR, 1), jnp.float32),
            pltpu.VMEM((_LR, 1), jnp.float32),
            pltpu.SMEM((1,), jnp.float32),
        ],
        interpret=INTERPRET,
    )(h1, h2, h2, w)
    return out[0, 0]


def _cnt_spec(ch):
    return pl.BlockSpec((1, _TCR, 16), lambda i, ch=ch: (ch, i, 0))


def _s_spec(ch):
    return pl.BlockSpec((1, _TCR, GCN_H), lambda i, ch=ch: (ch, i, 0))


def _full(shape):
    return pl.BlockSpec(shape, lambda i: tuple(0 for _ in shape))


_row64 = pl.BlockSpec((_TCR, GCN_H), lambda i: (i, 0))
_grid16 = (N_PAD // _TCR,)


def _prep(cnt, x1, x2, W11, W21):
    return pl.pallas_call(
        _prep_body,
        grid=_grid16,
        in_specs=[
            _cnt_spec(0), _cnt_spec(1),
            pl.BlockSpec((_TCR, IN_DIM), lambda i: (i, 0)),
            pl.BlockSpec((_TCR, IN_DIM), lambda i: (i, 0)),
            _full((IN_DIM, GCN_H)), _full((IN_DIM, GCN_H)),
        ],
        out_specs=[_row64, _row64],
        out_shape=[jax.ShapeDtypeStruct((N_PAD, GCN_H), jnp.float32)] * 2,
        interpret=INTERPRET,
    )(cnt, cnt, x1, x2, W11, W21)


def _mid(cnt, s, b11, b21, W12, W22):
    return pl.pallas_call(
        _mid_body,
        grid=_grid16,
        in_specs=[
            _cnt_spec(0), _cnt_spec(1), _s_spec(0), _s_spec(1),
            _full((1, GCN_H)), _full((1, GCN_H)),
            _full((GCN_H, GCN_H)), _full((GCN_H, GCN_H)),
        ],
        out_specs=[_row64, _row64],
        out_shape=[jax.ShapeDtypeStruct((N_PAD, GCN_H), jnp.float32)] * 2,
        interpret=INTERPRET,
    )(cnt, cnt, s, s, b11.reshape(1, -1), b21.reshape(1, -1), W12, W22)


def _fin(cnt, s, b12, b22, Wa1, ba1, Wa2, ba2, Wc1, bc1, Wc2, bc2):
    return pl.pallas_call(
        _fin_body,
        grid=_grid16,
        in_specs=[
            _cnt_spec(0), _cnt_spec(1), _s_spec(0), _s_spec(1),
            _full((1, GCN_H)), _full((1, GCN_H)),
            _full((2 * GCN_H, 32)), _full((1, 32)),
            _full((32, 2)), _full((1, 2)),
            _full((2 * GCN_H, 16)), _full((1, 16)),
            _full((16, 2)), _full((1, 2)),
        ],
        out_specs=[_row64, _row64, pl.BlockSpec((_TCR, 2), lambda i: (i, 0))],
        out_shape=[
            jax.ShapeDtypeStruct((N_NODES, GCN_H), jnp.float32),
            jax.ShapeDtypeStruct((N_NODES, GCN_H), jnp.float32),
            jax.ShapeDtypeStruct((N_NODES, 2), jnp.float32),
        ],
        interpret=INTERPRET,
    )(cnt, cnt, s, s, b12.reshape(1, -1), b22.reshape(1, -1),
      Wa1, ba1.reshape(1, -1), Wa2, ba2.reshape(1, -1),
      Wc1, bc1.reshape(1, -1), Wc2, bc2.reshape(1, -1))


def _class_weights(y):
    return pl.pallas_call(
        _weights_body,
        out_shape=jax.ShapeDtypeStruct((N_NODES, 1), jnp.float32),
        interpret=INTERPRET,
    )(y.reshape(N_NODES, 1))


def _pad_edges(edge_index):
    pad = jnp.full((E_PAD - N_EDGES,), DUMMY, jnp.int32)
    src = jnp.concatenate([edge_index[0].astype(jnp.int32), pad])
    dst = jnp.concatenate([edge_index[1].astype(jnp.int32), pad])
    return jnp.stack([src, dst])


def kernel(x1, edge_index1, x2, edge_index2, y,
           W11, b11, W12, b12, W21, b21, W22, b22,
           Wa1, ba1, Wa2, ba2, Wc1, bc1, Wc2, bc2):
    se = jnp.stack([_pad_edges(edge_index1), _pad_edges(edge_index2)])
    zeros16 = jnp.zeros((N_PAD, 16), jnp.float32)
    ones16 = jnp.ones((EK, 16), jnp.float32)

    cnt = _sc_counts(se, zeros16, ones16)
    h1p, h2p = _prep(cnt, x1, x2, W11, W21)
    s_l1 = _sc_conv(h1p, h2p, se)
    h1q, h2q = _mid(cnt, s_l1, b11, b21, W12, W22)
    s_l2 = _sc_conv(h1q, h2q, se)
    g1, g2, log_probs = _fin(cnt, s_l2, b12, b22,
                             Wa1, ba1, Wa2, ba2, Wc1, bc1, Wc2, bc2)
    w = _class_weights(y)
    loss = _contrastive_loss(g1, g2, w)
    return (log_probs, loss)
